# baseline (device time: 93196 ns/iter reference)
import jax
import jax.numpy as jnp
from jax import lax
from jax.experimental import pallas as pl
from jax.experimental.pallas import tpu as pltpu

N_DEV = 4
B_PER = 2
HQ_PER = 8
SQ = 512
SKV = 512
DH = 64
DMODEL = 768
HDIM = HQ_PER * DH
BLK = 64

_CompilerParams = getattr(pltpu, "CompilerParams", None) or getattr(
    pltpu, "TPUCompilerParams"
)


def kernel(x, Wq, K_ext, V_ext, Wo):
    bf16 = jnp.bfloat16
    my = lax.axis_index("i")

    K_my = lax.dynamic_slice_in_dim(K_ext, my * B_PER, B_PER, axis=0)
    V_my = lax.dynamic_slice_in_dim(V_ext, my * B_PER, B_PER, axis=0)
    kt = jnp.transpose(K_my, (0, 2, 1, 3)).astype(bf16)
    vt = jnp.transpose(V_my, (0, 2, 1, 3)).astype(bf16)

    x_bf = x.astype(bf16)
    payload = jnp.stack([Wq.astype(bf16), Wo.T.astype(bf16)])

    def body(
        x_ref, kt_ref, vt_ref, pay_ref, out_ref,
        comm_ref, q_ref, ctx_ref, bias_ref, send_sems, recv_sems,
    ):
        my_i = lax.axis_index("i")
        left = lax.rem(my_i + N_DEV - 1, N_DEV)
        right = lax.rem(my_i + 1, N_DEV)

        row = lax.broadcasted_iota(jnp.int32, (SQ, SKV), 0)
        col = lax.broadcasted_iota(jnp.int32, (SQ, SKV), 1)
        bias_ref[...] = jnp.where(
            col // BLK <= row // BLK, 0.0, -1e9
        ).astype(jnp.float32)

        comm_ref[0] = pay_ref[...]

        barrier_sem = pltpu.get_barrier_semaphore()
        for nbr in (left, right):
            pl.semaphore_signal(
                barrier_sem, inc=1,
                device_id=(nbr,), device_id_type=pl.DeviceIdType.MESH,
            )
        pl.semaphore_wait(barrier_sem, 2)

        for s in range(N_DEV):
            if s < N_DEV - 1:
                rdma = pltpu.make_async_remote_copy(
                    src_ref=comm_ref.at[s],
                    dst_ref=comm_ref.at[s + 1],
                    send_sem=send_sems.at[s],
                    recv_sem=recv_sems.at[s],
                    device_id=(right,),
                    device_id_type=pl.DeviceIdType.MESH,
                )
                rdma.start()

            j = lax.rem(my_i - s + N_DEV, N_DEV)
            hbase = j * HQ_PER
            wq = comm_ref[s, 0]
            woT = comm_ref[s, 1]

            for b in range(B_PER):
                q_ref[...] = lax.dot_general(
                    x_ref[b], wq, (((1,), (0,)), ((), ())),
                    preferred_element_type=jnp.float32,
                ).astype(bf16)
                for h in range(HQ_PER):
                    k_h = kt_ref[b, hbase + h]
                    v_h = vt_ref[b, hbase + h]
                    scores = lax.dot_general(
                        q_ref[:, h * DH:(h + 1) * DH], k_h,
                        (((1,), (1,)), ((), ())),
                        preferred_element_type=jnp.float32,
                    )
                    scores = scores * 0.125 + bias_ref[...]
                    m = jnp.max(scores, axis=-1, keepdims=True)
                    e = jnp.exp(scores - m)
                    w = (e / jnp.sum(e, axis=-1, keepdims=True)).astype(bf16)
                    ctx_ref[:, h * DH:(h + 1) * DH] = lax.dot_general(
                        w, v_h, (((1,), (0,)), ((), ())),
                        preferred_element_type=jnp.float32,
                    ).astype(bf16)
                part = lax.dot_general(
                    ctx_ref[...], woT, (((1,), (1,)), ((), ())),
                    preferred_element_type=jnp.float32,
                )
                if s == 0:
                    out_ref[b] = part
                else:
                    out_ref[b] = out_ref[b] + part

            if s < N_DEV - 1:
                rdma.wait()

    return pl.pallas_call(
        body,
        out_shape=jax.ShapeDtypeStruct((B_PER, SQ, DMODEL), jnp.float32),
        in_specs=[pl.BlockSpec(memory_space=pltpu.VMEM)] * 4,
        out_specs=pl.BlockSpec(memory_space=pltpu.VMEM),
        scratch_shapes=[
            pltpu.VMEM((N_DEV, 2, DMODEL, HDIM), bf16),
            pltpu.VMEM((SQ, HDIM), bf16),
            pltpu.VMEM((SQ, HDIM), bf16),
            pltpu.VMEM((SQ, SKV), jnp.float32),
            pltpu.SemaphoreType.DMA((N_DEV - 1,)),
            pltpu.SemaphoreType.DMA((N_DEV - 1,)),
        ],
        compiler_params=_CompilerParams(collective_id=0),
    )(x_bf, kt, vt, payload)


# device time: 80214 ns/iter; 1.1618x vs baseline; 1.1618x over previous
import jax
import jax.numpy as jnp
from jax import lax
from jax.experimental import pallas as pl
from jax.experimental.pallas import tpu as pltpu

N_DEV = 4
B_PER = 2
HQ_PER = 8
SQ = 512
SKV = 512
DH = 64
DMODEL = 768
HDIM = HQ_PER * DH
BLK = 64

_CompilerParams = getattr(pltpu, "CompilerParams", None) or getattr(
    pltpu, "TPUCompilerParams"
)


def kernel(x, Wq, K_ext, V_ext, Wo):
    bf16 = jnp.bfloat16
    my = lax.axis_index("i")

    K_my = lax.dynamic_slice_in_dim(K_ext, my * B_PER, B_PER, axis=0)
    V_my = lax.dynamic_slice_in_dim(V_ext, my * B_PER, B_PER, axis=0)
    kt = jnp.transpose(K_my, (0, 2, 1, 3)).astype(bf16)
    vt = jnp.transpose(V_my, (0, 2, 1, 3)).astype(bf16)

    x_bf = x.astype(bf16)
    payload = jnp.stack([Wq.astype(bf16), Wo.T.astype(bf16)])

    def body(
        x_ref, kt_ref, vt_ref, pay_ref, out_ref,
        comm_ref, q_ref, ctx_ref, bias_ref, send_sems, recv_sems,
    ):
        my_i = lax.axis_index("i")
        left = lax.rem(my_i + N_DEV - 1, N_DEV)
        right = lax.rem(my_i + 1, N_DEV)

        row = lax.broadcasted_iota(jnp.int32, (SQ, SKV), 0)
        col = lax.broadcasted_iota(jnp.int32, (SQ, SKV), 1)
        bias_ref[...] = jnp.where(
            col // BLK <= row // BLK, 0.0, -1e9
        ).astype(jnp.float32)

        comm_ref[0] = pay_ref[...]

        barrier_sem = pltpu.get_barrier_semaphore()
        for nbr in (left, right):
            pl.semaphore_signal(
                barrier_sem, inc=1,
                device_id=(nbr,), device_id_type=pl.DeviceIdType.MESH,
            )
        pl.semaphore_wait(barrier_sem, 2)

        def remote_copy(src, dst, sem_idx, dev):
            return pltpu.make_async_remote_copy(
                src_ref=src, dst_ref=dst,
                send_sem=send_sems.at[sem_idx],
                recv_sem=recv_sems.at[sem_idx],
                device_id=(dev,),
                device_id_type=pl.DeviceIdType.MESH,
            )

        rdma_a = remote_copy(comm_ref.at[0], comm_ref.at[1], 0, right)
        rdma_b = remote_copy(comm_ref.at[0], comm_ref.at[2], 1, left)
        rdma_a.start()
        rdma_b.start()

        def compute(s, j, first=False):
            hbase = j * HQ_PER
            wq = comm_ref[s, 0]
            woT = comm_ref[s, 1]
            for b in range(B_PER):
                q_ref[...] = lax.dot_general(
                    x_ref[b], wq, (((1,), (0,)), ((), ())),
                    preferred_element_type=jnp.float32,
                ).astype(bf16)
                for h in range(HQ_PER):
                    k_h = kt_ref[b, hbase + h]
                    v_h = vt_ref[b, hbase + h]
                    scores = lax.dot_general(
                        q_ref[:, h * DH:(h + 1) * DH], k_h,
                        (((1,), (1,)), ((), ())),
                        preferred_element_type=jnp.float32,
                    )
                    scores = scores * 0.125 + bias_ref[...]
                    m = jnp.max(scores, axis=-1, keepdims=True)
                    e = jnp.exp(scores - m)
                    w = (e / jnp.sum(e, axis=-1, keepdims=True)).astype(bf16)
                    ctx_ref[:, h * DH:(h + 1) * DH] = lax.dot_general(
                        w, v_h, (((1,), (0,)), ((), ())),
                        preferred_element_type=jnp.float32,
                    ).astype(bf16)
                part = lax.dot_general(
                    ctx_ref[...], woT, (((1,), (1,)), ((), ())),
                    preferred_element_type=jnp.float32,
                )
                if first:
                    out_ref[b] = part
                else:
                    out_ref[b] = out_ref[b] + part

        compute(0, my_i, first=True)

        rdma_a.wait_recv()
        rdma_b.wait_recv()
        rdma_c = remote_copy(comm_ref.at[1, 0], comm_ref.at[3, 0], 2, right)
        rdma_d = remote_copy(comm_ref.at[2, 1], comm_ref.at[3, 1], 3, left)
        rdma_c.start()
        rdma_d.start()

        compute(1, left)
        compute(2, right)

        rdma_c.wait_recv()
        rdma_d.wait_recv()
        compute(3, lax.rem(my_i + 2, N_DEV))

        rdma_a.wait_send()
        rdma_b.wait_send()
        rdma_c.wait_send()
        rdma_d.wait_send()

    return pl.pallas_call(
        body,
        out_shape=jax.ShapeDtypeStruct((B_PER, SQ, DMODEL), jnp.float32),
        in_specs=[pl.BlockSpec(memory_space=pltpu.VMEM)] * 4,
        out_specs=pl.BlockSpec(memory_space=pltpu.VMEM),
        scratch_shapes=[
            pltpu.VMEM((N_DEV, 2, DMODEL, HDIM), bf16),
            pltpu.VMEM((SQ, HDIM), bf16),
            pltpu.VMEM((SQ, HDIM), bf16),
            pltpu.VMEM((SQ, SKV), jnp.float32),
            pltpu.SemaphoreType.DMA((4,)),
            pltpu.SemaphoreType.DMA((4,)),
        ],
        compiler_params=_CompilerParams(collective_id=0),
    )(x_bf, kt, vt, payload)


# device time: 67381 ns/iter; 1.3831x vs baseline; 1.1905x over previous
import jax
import jax.numpy as jnp
from jax import lax
from jax.experimental import pallas as pl
from jax.experimental.pallas import tpu as pltpu

N_DEV = 4
B_PER = 2
HQ_PER = 8
SQ = 512
SKV = 512
DH = 64
DMODEL = 768
HDIM = HQ_PER * DH
BLK = 64

_CompilerParams = getattr(pltpu, "CompilerParams", None) or getattr(
    pltpu, "TPUCompilerParams"
)


def kernel(x, Wq, K_ext, V_ext, Wo):
    bf16 = jnp.bfloat16
    my = lax.axis_index("i")

    K_my = lax.dynamic_slice_in_dim(K_ext, my * B_PER, B_PER, axis=0)
    V_my = lax.dynamic_slice_in_dim(V_ext, my * B_PER, B_PER, axis=0)
    kt = jnp.transpose(K_my, (0, 2, 1, 3)).astype(bf16)
    vt = jnp.transpose(V_my, (0, 2, 1, 3)).astype(bf16)

    x_bf = x.astype(bf16)
    payload = jnp.stack([Wq.astype(bf16), Wo.T.astype(bf16)])

    def body(
        x_ref, kt_ref, vt_ref, pay_ref, out_ref,
        comm_ref, q_ref, ctx_ref, bias_ref, send_sems, recv_sems,
    ):
        my_i = lax.axis_index("i")
        left = lax.rem(my_i + N_DEV - 1, N_DEV)
        right = lax.rem(my_i + 1, N_DEV)

        row = lax.broadcasted_iota(jnp.int32, (SQ, SKV), 0)
        col = lax.broadcasted_iota(jnp.int32, (SQ, SKV), 1)
        bias_ref[...] = jnp.where(
            col // BLK <= row // BLK, 0.0, -1e9
        ).astype(jnp.float32)

        comm_ref[0] = pay_ref[...]

        barrier_sem = pltpu.get_barrier_semaphore()
        for nbr in (left, right):
            pl.semaphore_signal(
                barrier_sem, inc=1,
                device_id=(nbr,), device_id_type=pl.DeviceIdType.MESH,
            )
        pl.semaphore_wait(barrier_sem, 2)

        def remote_copy(src, dst, sem_idx, dev):
            return pltpu.make_async_remote_copy(
                src_ref=src, dst_ref=dst,
                send_sem=send_sems.at[sem_idx],
                recv_sem=recv_sems.at[sem_idx],
                device_id=(dev,),
                device_id_type=pl.DeviceIdType.MESH,
            )

        rdma_a = remote_copy(comm_ref.at[0], comm_ref.at[1], 0, right)
        rdma_b = remote_copy(comm_ref.at[0], comm_ref.at[2], 1, left)
        rdma_a.start()
        rdma_b.start()

        def compute(s, j, first=False):
            hbase = j * HQ_PER
            wq = comm_ref[s, 0]
            woT = comm_ref[s, 1]
            for b in range(B_PER):
                q_ref[...] = lax.dot_general(
                    x_ref[b], wq, (((1,), (0,)), ((), ())),
                    preferred_element_type=jnp.float32,
                ).astype(bf16)
                for h in range(HQ_PER):
                    k_h = kt_ref[b, hbase + h]
                    v_h = vt_ref[b, hbase + h]
                    for lo, hi, nk in ((0, SQ // 2, SKV // 2),
                                       (SQ // 2, SQ, SKV)):
                        scores = lax.dot_general(
                            q_ref[lo:hi, h * DH:(h + 1) * DH], k_h[:nk],
                            (((1,), (1,)), ((), ())),
                            preferred_element_type=jnp.float32,
                        )
                        e = jnp.exp(
                            scores * 0.125 + bias_ref[lo:hi, :nk]
                        ).astype(bf16)
                        s = lax.dot_general(
                            e, v_h[:nk], (((1,), (0,)), ((), ())),
                            preferred_element_type=jnp.float32,
                        )
                        denom = jnp.sum(
                            e.astype(jnp.float32), axis=-1, keepdims=True
                        )
                        ctx_ref[lo:hi, h * DH:(h + 1) * DH] = (
                            s / denom
                        ).astype(bf16)
                part = lax.dot_general(
                    ctx_ref[...], woT, (((1,), (1,)), ((), ())),
                    preferred_element_type=jnp.float32,
                )
                if first:
                    out_ref[b] = part
                else:
                    out_ref[b] = out_ref[b] + part

        compute(0, my_i, first=True)

        rdma_a.wait_recv()
        rdma_b.wait_recv()
        rdma_c = remote_copy(comm_ref.at[1, 0], comm_ref.at[3, 0], 2, right)
        rdma_d = remote_copy(comm_ref.at[2, 1], comm_ref.at[3, 1], 3, left)
        rdma_c.start()
        rdma_d.start()

        compute(1, left)
        compute(2, right)

        rdma_c.wait_recv()
        rdma_d.wait_recv()
        compute(3, lax.rem(my_i + 2, N_DEV))

        rdma_a.wait_send()
        rdma_b.wait_send()
        rdma_c.wait_send()
        rdma_d.wait_send()

    return pl.pallas_call(
        body,
        out_shape=jax.ShapeDtypeStruct((B_PER, SQ, DMODEL), jnp.float32),
        in_specs=[pl.BlockSpec(memory_space=pltpu.VMEM)] * 4,
        out_specs=pl.BlockSpec(memory_space=pltpu.VMEM),
        scratch_shapes=[
            pltpu.VMEM((N_DEV, 2, DMODEL, HDIM), bf16),
            pltpu.VMEM((SQ, HDIM), bf16),
            pltpu.VMEM((SQ, HDIM), bf16),
            pltpu.VMEM((SQ, SKV), jnp.float32),
            pltpu.SemaphoreType.DMA((4,)),
            pltpu.SemaphoreType.DMA((4,)),
        ],
        compiler_params=_CompilerParams(collective_id=0),
    )(x_bf, kt, vt, payload)
